# baseline (speedup 1.0000x reference)
import jax
import jax.numpy as jnp
import numpy as np
from jax import lax
from jax.experimental import pallas as pl
from jax.experimental.pallas import tpu as pltpu
from jax.experimental.pallas import tpu_sc as plsc

N_USERS = 25000
N_ITEMS = 25000
N_NODES = N_USERS + N_ITEMS
E_EDGES = 800000
DIM = 64
DH = 32
LEAKY_SLOPE = 0.2
NC = 2
NS = 16
CHUNK = 128
KPT = 400
KC = KPT * NS
EP = KC * CHUNK
BLK = 4
NBLK = KPT // BLK
PAIRS = NBLK // 2
NBT = KC // BLK
WCH = 80
NWC = N_NODES // WCH
NWT = (NWC + NS - 1) // NS


_GD = lax.GatherDimensionNumbers(offset_dims=(), collapsed_slice_dims=(0,),
                                 start_index_map=(0,))


def _bcast16(vec, e):
  idx = jnp.full((16, 1), e, dtype=jnp.int32)
  return lax.gather(vec, idx, _GD, (1,),
                    mode=lax.GatherScatterMode.PROMISE_IN_BOUNDS)


def _prop_body(ego0, pk1, pk2,
               ego_out, ego2, tbuf,
               accum, idx_b, rv, sv, acc_v, prev_v, wb, zero_v,
               gsem, ssem):
  c = lax.axis_index("c")
  s = lax.axis_index("s")

  def zrow(i, _):
    zero_v[i, pl.ds(0, 16)] = jnp.zeros((16,), jnp.float32)
    zero_v[i, pl.ds(16, 16)] = jnp.zeros((16,), jnp.float32)
    return 0
  lax.fori_loop(0, WCH, zrow, 0)

  def init_w(w, _):
    cid = w * NS + s
    @pl.when(cid < NWC)
    def _():
      rb = cid * WCH
      pltpu.sync_copy(zero_v, accum.at[pl.ds(rb, WCH)])
      pltpu.sync_copy(ego0.at[pl.ds(c * N_NODES + rb, WCH)], wb)
      pltpu.sync_copy(wb, ego2.at[pl.ds(c * N_NODES + rb, WCH)])
    return 0
  lax.fori_loop(0, NWT, init_w, 0)
  plsc.subcore_barrier()

  def idx_fix(qn):
    coff = jnp.full((16,), c * N_NODES, dtype=jnp.int32)
    for r in range(BLK):
      for g in range(8):
        sl = pl.ds(g * 16, 16)
        idx_b[qn * 12 + r, sl] = idx_b[qn * 12 + r, sl] + coff

  def spmm(gtab, pk):
    tb = s * NBLK
    pltpu.sync_copy(pk.at[tb], idx_b.at[pl.ds(0, 12)])
    idx_fix(0)
    pltpu.async_copy(gtab.at[idx_b.at[0]], rv.at[0], gsem)
    pltpu.async_copy(gtab.at[idx_b.at[1]], rv.at[1], gsem)

    def step(cj, _):
      blk = cj // 4
      q = lax.rem(blk, 2)
      r = lax.rem(cj, 4)
      @pl.when(cj >= 2)
      def _():
        cp = cj - 2
        qp = lax.rem(cp // 4, 2)
        pltpu.make_async_copy(sv.at[lax.rem(cp, 2)],
                              accum.at[idx_b.at[qp * 12 + 4 + lax.rem(cp, 4)]],
                              ssem).wait()
      @pl.when((r == 1) & (blk + 1 < NBLK))
      def _():
        qn = lax.rem(blk + 1, 2)
        pltpu.sync_copy(pk.at[tb + blk + 1], idx_b.at[pl.ds(qn * 12, 12)])
        idx_fix(qn)
      @pl.when(cj + 2 < KPT)
      def _():
        cn = cj + 2
        rn = lax.rem(cn, 4)
        qn2 = lax.rem(cn // 4, 2)
        pltpu.async_copy(gtab.at[idx_b.at[qn2 * 12 + rn]], rv.at[rn], gsem)
      pltpu.make_async_copy(gtab.at[idx_b.at[q * 12 + r]], rv.at[r],
                            gsem).wait()
      vrow = q * 12 + 8 + r
      sr = lax.rem(cj, 2)
      for g in range(8):
        vv = plsc.bitcast(idx_b[vrow, pl.ds(g * 16, 16)], jnp.float32)
        for e in range(16):
          i = g * 16 + e
          vb = _bcast16(vv, e)
          a, b = plsc.unpack(rv[r, i, :], format=plsc.PackFormat.INTERLEAVED)
          sv[sr, i, pl.ds(0, 16)] = a * vb
          sv[sr, i, pl.ds(16, 16)] = b * vb
      pltpu.async_copy(sv.at[sr], accum.at[idx_b.at[q * 12 + 4 + r]], ssem,
                       add=True)
      return 0
    lax.fori_loop(0, KPT, step, 0)
    pltpu.make_async_copy(sv.at[0], accum.at[idx_b.at[12 + 4 + 2]],
                          ssem).wait()
    pltpu.make_async_copy(sv.at[1], accum.at[idx_b.at[12 + 4 + 3]],
                          ssem).wait()

  def writeout_t():
    def w(wi, _):
      cid = wi * NS + s
      @pl.when(cid < NWC)
      def _():
        rb = cid * WCH
        pltpu.sync_copy(accum.at[pl.ds(rb, WCH)], acc_v)

        def rowt(i, _):
          wb[i, :] = plsc.pack(acc_v[i, pl.ds(0, 16)], acc_v[i, pl.ds(16, 16)],
                               format=plsc.PackFormat.INTERLEAVED)
          return 0
        lax.fori_loop(0, WCH, rowt, 0)
        pltpu.sync_copy(wb, tbuf.at[pl.ds(c * N_NODES + rb, WCH)])
        pltpu.sync_copy(zero_v, accum.at[pl.ds(rb, WCH)])
      return 0
    lax.fori_loop(0, NWT, w, 0)

  def writeout_ego(k):
    slope = jnp.where(k == 1, 1.0, LEAKY_SLOPE).astype(jnp.float32)
    slv = jnp.full((16,), slope, dtype=jnp.float32)

    def w(wi, _):
      cid = wi * NS + s
      @pl.when(cid < NWC)
      def _():
        rb = cid * WCH
        hb = c * N_NODES + rb
        pltpu.sync_copy(accum.at[pl.ds(rb, WCH)], acc_v)
        pltpu.sync_copy(ego2.at[pl.ds(hb, WCH)], prev_v)

        def rowf(i, _):
          pa, pb = plsc.unpack(prev_v[i, :],
                               format=plsc.PackFormat.INTERLEAVED)
          x0 = acc_v[i, pl.ds(0, 16)]
          x1 = acc_v[i, pl.ds(16, 16)]
          x0 = jnp.where(x0 >= 0, x0, x0 * slv) + pa
          x1 = jnp.where(x1 >= 0, x1, x1 * slv) + pb
          acc_v[i, pl.ds(0, 16)] = x0
          acc_v[i, pl.ds(16, 16)] = x1
          wb[i, :] = plsc.pack(x0, x1, format=plsc.PackFormat.INTERLEAVED)
          return 0
        lax.fori_loop(0, WCH, rowf, 0)

        pltpu.sync_copy(wb, ego2.at[pl.ds(hb, WCH)])
        pltpu.sync_copy(acc_v, ego_out.at[pl.ds(hb, WCH)])
        pltpu.sync_copy(zero_v, accum.at[pl.ds(rb, WCH)])
      return 0
    lax.fori_loop(0, NWT, w, 0)

  def layer(k, _):
    spmm(ego2, pk1)
    plsc.subcore_barrier()
    writeout_t()
    plsc.subcore_barrier()
    spmm(tbuf, pk2)
    plsc.subcore_barrier()
    writeout_ego(k)
    plsc.subcore_barrier()
    return 0
  lax.fori_loop(0, 2, layer, 0)


_TABF = jax.ShapeDtypeStruct((2 * N_NODES, DH), jnp.float32)
_TABH = jax.ShapeDtypeStruct((2 * N_NODES, DH), jnp.bfloat16)
_propagate = pl.kernel(
    _prop_body,
    out_type=(_TABF, _TABH, _TABH),
    mesh=plsc.VectorSubcoreMesh(core_axis_name="c", subcore_axis_name="s"),
    compiler_params=pltpu.CompilerParams(use_tc_tiling_on_sc=False,
                                         needs_layout_passes=False),
    scratch_types=[
        pltpu.VMEM_SHARED((N_NODES, DH), jnp.float32),
        pltpu.VMEM((24, CHUNK), jnp.int32),
        pltpu.VMEM((4, CHUNK, DH), jnp.bfloat16),
        pltpu.VMEM((2, CHUNK, DH), jnp.float32),
        pltpu.VMEM((WCH, DH), jnp.float32),
        pltpu.VMEM((WCH, DH), jnp.bfloat16),
        pltpu.VMEM((WCH, DH), jnp.bfloat16),
        pltpu.VMEM((WCH, DH), jnp.float32),
        pltpu.SemaphoreType.DMA,
        pltpu.SemaphoreType.DMA,
    ],
)


def _mm_body(x_ref, w_ref, o_ref):
  hb = pl.program_id(0)
  ib = pl.program_id(1)
  whu = jnp.where(hb == 0, w_ref[0][:, :DH], w_ref[0][:, DH:])
  whi = jnp.where(hb == 0, w_ref[1][:, :DH], w_ref[1][:, DH:])
  ou = jnp.dot(x_ref[...], whu, preferred_element_type=jnp.float32)
  oi = jnp.dot(x_ref[...], whi, preferred_element_type=jnp.float32)
  rid = ib * 2000 + lax.broadcasted_iota(jnp.int32, (2000, 1), 0)
  o_ref[0] = jnp.where(rid < N_USERS, ou, oi).astype(jnp.bfloat16)


_ILV = np.stack([np.arange(16), np.arange(16) + 16], axis=1).reshape(-1)


def _ego_flat(user_emb, item_emb, user_w, item_w):
  x = jnp.concatenate([user_emb, item_emb], axis=0)

  def pw(w):
    return jnp.concatenate([w[:, :DH][:, _ILV], w[:, DH:][:, _ILV]], axis=1)
  w2 = jnp.stack([pw(user_w), pw(item_w)])
  bm = 2000
  out = pl.pallas_call(
      _mm_body,
      grid=(2, N_NODES // bm),
      in_specs=[
          pl.BlockSpec((bm, DIM), lambda h, i: (i, 0)),
          pl.BlockSpec((2, DIM, DIM), lambda h, i: (0, 0, 0)),
      ],
      out_specs=pl.BlockSpec((1, bm, DH), lambda h, i: (h, i, 0)),
      out_shape=jax.ShapeDtypeStruct((2, N_NODES, DH), jnp.bfloat16),
  )(x, w2)
  return out.reshape(2 * N_NODES, DH)


def _pack(src, dst, vals):
  vi = lax.bitcast_convert_type(vals, jnp.int32)
  s3 = src.reshape(NBT, BLK, CHUNK)
  d3 = dst.reshape(NBT, BLK, CHUNK)
  v3 = vi.reshape(NBT, BLK, CHUNK)
  return jnp.concatenate([s3, d3, v3], axis=1)


def kernel(user_emb, item_emb, user_w, item_w, adj_values, adj_indices,
           keep_rate=1):
  ego_flat = _ego_flat(user_emb, item_emb, user_w, item_w)

  rows = adj_indices[0].astype(jnp.int32)
  cols = adj_indices[1].astype(jnp.int32)
  pad = EP - E_EDGES
  pad_idx = (jnp.arange(pad, dtype=jnp.int32) * 61) % N_NODES
  rows_p = jnp.concatenate([rows, pad_idx])
  cols_p = jnp.concatenate([cols, pad_idx])
  vals_p = jnp.concatenate([adj_values, jnp.zeros((pad,), jnp.float32)])

  pk1 = _pack(rows_p, cols_p, vals_p)
  pk2 = _pack(cols_p, rows_p, vals_p)

  ego_out, _, _ = _propagate(ego_flat, pk1, pk2)
  full = jnp.concatenate([ego_out[:N_NODES], ego_out[N_NODES:]], axis=1)
  return full[:N_USERS], full[N_USERS:]


# speedup vs baseline: 1.4913x; 1.4913x over previous
import jax
import jax.numpy as jnp
from jax import lax
from jax.experimental import pallas as pl
from jax.experimental.pallas import tpu as pltpu
from jax.experimental.pallas import tpu_sc as plsc

N_USERS = 25000
N_ITEMS = 25000
N_NODES = N_USERS + N_ITEMS
E_EDGES = 800000
DIM = 64
DH = 32
LEAKY_SLOPE = 0.2
NC = 2
NS = 16
CHUNK = 128
KPT = 400
KC = KPT * NS
EP = KC * CHUNK
BLK = 4
NBLK = KPT // BLK
PAIRS = NBLK // 2
NBT = KC // BLK
WCH = 80
NWC = N_NODES // WCH
NWT = (NWC + NS - 1) // NS


_GD = lax.GatherDimensionNumbers(offset_dims=(), collapsed_slice_dims=(0,),
                                 start_index_map=(0,))


def _bcast16(vec, e):
  idx = jnp.full((16, 1), e, dtype=jnp.int32)
  return lax.gather(vec, idx, _GD, (1,),
                    mode=lax.GatherScatterMode.PROMISE_IN_BOUNDS)


def _prop_body(ego0, pk1, pk2,
               ego2, tbuf,
               accum, idx_b, rv, acc_v, prev_v, zero_v,
               gsem, ssem):
  c = lax.axis_index("c")
  s = lax.axis_index("s")

  def zrow(i, _):
    zero_v[i, pl.ds(0, 16)] = jnp.zeros((16,), jnp.float32)
    zero_v[i, pl.ds(16, 16)] = jnp.zeros((16,), jnp.float32)
    return 0
  lax.fori_loop(0, WCH, zrow, 0)

  def init_w(w, _):
    cid = w * NS + s
    @pl.when(cid < NWC)
    def _():
      rb = cid * WCH
      pltpu.sync_copy(zero_v, accum.at[pl.ds(rb, WCH)])
      pltpu.sync_copy(ego0.at[pl.ds(c * N_NODES + rb, WCH)], acc_v)
      pltpu.sync_copy(acc_v, ego2.at[pl.ds(c * N_NODES + rb, WCH)])
    return 0
  lax.fori_loop(0, NWT, init_w, 0)
  plsc.subcore_barrier()

  def idx_fix(qn):
    coff = jnp.full((16,), c * N_NODES, dtype=jnp.int32)
    for r in range(BLK):
      for g in range(8):
        sl = pl.ds(g * 16, 16)
        idx_b[qn * 12 + r, sl] = idx_b[qn * 12 + r, sl] + coff

  def spmm(gtab, pk):
    tb = s * NBLK
    pltpu.sync_copy(pk.at[tb], idx_b.at[pl.ds(0, 12)])
    idx_fix(0)
    pltpu.async_copy(gtab.at[idx_b.at[0]], rv.at[0], gsem)
    pltpu.async_copy(gtab.at[idx_b.at[1]], rv.at[1], gsem)

    def step(cj, _):
      blk = cj // 4
      q = lax.rem(blk, 2)
      r = lax.rem(cj, 4)
      @pl.when(cj >= 2)
      def _():
        cp = cj - 2
        rp = lax.rem(cp, 4)
        qp = lax.rem(cp // 4, 2)
        pltpu.make_async_copy(rv.at[rp], accum.at[idx_b.at[qp * 12 + 4 + rp]],
                              ssem).wait()
      @pl.when((r == 1) & (blk + 1 < NBLK))
      def _():
        qn = lax.rem(blk + 1, 2)
        pltpu.sync_copy(pk.at[tb + blk + 1], idx_b.at[pl.ds(qn * 12, 12)])
        idx_fix(qn)
      @pl.when(cj + 2 < KPT)
      def _():
        cn = cj + 2
        rn = lax.rem(cn, 4)
        qn2 = lax.rem(cn // 4, 2)
        pltpu.async_copy(gtab.at[idx_b.at[qn2 * 12 + rn]], rv.at[rn], gsem)
      pltpu.make_async_copy(gtab.at[idx_b.at[q * 12 + r]], rv.at[r],
                            gsem).wait()
      vrow = q * 12 + 8 + r
      for g in range(8):
        vv = plsc.bitcast(idx_b[vrow, pl.ds(g * 16, 16)], jnp.float32)
        for e in range(16):
          i = g * 16 + e
          vb = _bcast16(vv, e)
          for h in range(2):
            sl = pl.ds(h * 16, 16)
            rv[r, i, sl] = rv[r, i, sl] * vb
      pltpu.async_copy(rv.at[r], accum.at[idx_b.at[q * 12 + 4 + r]], ssem,
                       add=True)
      return 0
    lax.fori_loop(0, KPT, step, 0)
    pltpu.make_async_copy(rv.at[2], accum.at[idx_b.at[12 + 4 + 2]],
                          ssem).wait()
    pltpu.make_async_copy(rv.at[3], accum.at[idx_b.at[12 + 4 + 3]],
                          ssem).wait()

  def writeout_t():
    def w(wi, _):
      cid = wi * NS + s
      @pl.when(cid < NWC)
      def _():
        rb = cid * WCH
        pltpu.sync_copy(accum.at[pl.ds(rb, WCH)],
                        tbuf.at[pl.ds(c * N_NODES + rb, WCH)])
        pltpu.sync_copy(zero_v, accum.at[pl.ds(rb, WCH)])
      return 0
    lax.fori_loop(0, NWT, w, 0)

  def writeout_ego(k):
    slope = jnp.where(k == 1, 1.0, LEAKY_SLOPE).astype(jnp.float32)
    sv = jnp.full((16,), slope, dtype=jnp.float32)

    def w(wi, _):
      cid = wi * NS + s
      @pl.when(cid < NWC)
      def _():
        rb = cid * WCH
        hb = c * N_NODES + rb
        pltpu.sync_copy(accum.at[pl.ds(rb, WCH)], acc_v)
        pltpu.sync_copy(ego2.at[pl.ds(hb, WCH)], prev_v)

        def rowf(i, _):
          for h in range(2):
            sl = pl.ds(h * 16, 16)
            x = acc_v[i, sl]
            x = jnp.where(x >= 0, x, x * sv)
            acc_v[i, sl] = x + prev_v[i, sl]
          return 0
        lax.fori_loop(0, WCH, rowf, 0)

        pltpu.sync_copy(acc_v, ego2.at[pl.ds(hb, WCH)])
        pltpu.sync_copy(zero_v, accum.at[pl.ds(rb, WCH)])
      return 0
    lax.fori_loop(0, NWT, w, 0)

  def layer(k, _):
    spmm(ego2, pk1)
    plsc.subcore_barrier()
    writeout_t()
    plsc.subcore_barrier()
    spmm(tbuf, pk2)
    plsc.subcore_barrier()
    writeout_ego(k)
    plsc.subcore_barrier()
    return 0
  lax.fori_loop(0, 2, layer, 0)


_TAB = jax.ShapeDtypeStruct((2 * N_NODES, DH), jnp.float32)
_propagate = pl.kernel(
    _prop_body,
    out_type=(_TAB, _TAB),
    mesh=plsc.VectorSubcoreMesh(core_axis_name="c", subcore_axis_name="s"),
    compiler_params=pltpu.CompilerParams(use_tc_tiling_on_sc=False,
                                         needs_layout_passes=False),
    scratch_types=[
        pltpu.VMEM_SHARED((N_NODES, DH), jnp.float32),
        pltpu.VMEM((24, CHUNK), jnp.int32),
        pltpu.VMEM((4, CHUNK, DH), jnp.float32),
        pltpu.VMEM((WCH, DH), jnp.float32),
        pltpu.VMEM((WCH, DH), jnp.float32),
        pltpu.VMEM((WCH, DH), jnp.float32),
        pltpu.SemaphoreType.DMA,
        pltpu.SemaphoreType.DMA,
    ],
)


def _mm_body(x_ref, w_ref, o_ref):
  hb = pl.program_id(0)
  ib = pl.program_id(1)
  w = jnp.where(ib < N_USERS // 1000, w_ref[0], w_ref[1])
  wh = jnp.where(hb == 0, w[:, :DH], w[:, DH:])
  o_ref[...] = jnp.dot(x_ref[...], wh, preferred_element_type=jnp.float32)


def _ego_flat(user_emb, item_emb, user_w, item_w):
  x = jnp.concatenate([user_emb, item_emb], axis=0)
  w2 = jnp.stack([user_w, item_w])
  bm = 1000
  return pl.pallas_call(
      _mm_body,
      grid=(2, N_NODES // bm),
      in_specs=[
          pl.BlockSpec((bm, DIM), lambda h, i: (i, 0)),
          pl.BlockSpec((2, DIM, DIM), lambda h, i: (0, 0, 0)),
      ],
      out_specs=pl.BlockSpec((bm, DH),
                             lambda h, i: (h * (N_NODES // bm) + i, 0)),
      out_shape=jax.ShapeDtypeStruct((2 * N_NODES, DH), jnp.float32),
  )(x, w2)


def _pack(src, dst, vals):
  vi = lax.bitcast_convert_type(vals, jnp.int32)
  s3 = src.reshape(NBT, BLK, CHUNK)
  d3 = dst.reshape(NBT, BLK, CHUNK)
  v3 = vi.reshape(NBT, BLK, CHUNK)
  return jnp.concatenate([s3, d3, v3], axis=1)


def kernel(user_emb, item_emb, user_w, item_w, adj_values, adj_indices,
           keep_rate=1):
  ego_flat = _ego_flat(user_emb, item_emb, user_w, item_w)

  rows = adj_indices[0].astype(jnp.int32)
  cols = adj_indices[1].astype(jnp.int32)
  pad = EP - E_EDGES
  pad_idx = (jnp.arange(pad, dtype=jnp.int32) * 61) % N_NODES
  rows_p = jnp.concatenate([rows, pad_idx])
  cols_p = jnp.concatenate([cols, pad_idx])
  vals_p = jnp.concatenate([adj_values, jnp.zeros((pad,), jnp.float32)])

  pk1 = _pack(rows_p, cols_p, vals_p)
  pk2 = _pack(cols_p, rows_p, vals_p)

  ego2, _ = _propagate(ego_flat, pk1, pk2)
  full = jnp.concatenate([ego2[:N_NODES], ego2[N_NODES:]], axis=1)
  return full[:N_USERS], full[N_USERS:]


# speedup vs baseline: 1.5159x; 1.0165x over previous
import jax
import jax.numpy as jnp
from jax import lax
from jax.experimental import pallas as pl
from jax.experimental.pallas import tpu as pltpu
from jax.experimental.pallas import tpu_sc as plsc

N_USERS = 25000
N_ITEMS = 25000
N_NODES = N_USERS + N_ITEMS
E_EDGES = 800000
DIM = 64
DH = 32
LEAKY_SLOPE = 0.2
NC = 2
NS = 16
CHUNK = 128
KPT = 400
KC = KPT * NS
EP = KC * CHUNK
BLK = 4
NBLK = KPT // BLK
PAIRS = NBLK // 2
NBT = KC // BLK
WCH = 80
NWC = N_NODES // WCH
NWT = (NWC + NS - 1) // NS


_GD = lax.GatherDimensionNumbers(offset_dims=(), collapsed_slice_dims=(0,),
                                 start_index_map=(0,))


def _bcast16(vec, e):
  idx = jnp.full((16, 1), e, dtype=jnp.int32)
  return lax.gather(vec, idx, _GD, (1,),
                    mode=lax.GatherScatterMode.PROMISE_IN_BOUNDS)


def _prop_body(ego0, pk1, pk2,
               ego2, tbuf,
               accum, idx_b, rv, acc_v, prev_v, zero_v,
               gsem, ssem):
  c = lax.axis_index("c")
  s = lax.axis_index("s")

  def zrow(i, _):
    zero_v[i, pl.ds(0, 16)] = jnp.zeros((16,), jnp.float32)
    zero_v[i, pl.ds(16, 16)] = jnp.zeros((16,), jnp.float32)
    return 0
  lax.fori_loop(0, WCH, zrow, 0)

  def init_w(w, _):
    cid = w * NS + s
    @pl.when(cid < NWC)
    def _():
      rb = cid * WCH
      pltpu.sync_copy(zero_v, accum.at[pl.ds(rb, WCH)])
      pltpu.sync_copy(ego0.at[pl.ds(c * N_NODES + rb, WCH)], acc_v)
      pltpu.sync_copy(acc_v, ego2.at[pl.ds(c * N_NODES + rb, WCH)])
    return 0
  lax.fori_loop(0, NWT, init_w, 0)
  plsc.subcore_barrier()

  def idx_fix(qn):
    coff = jnp.full((16,), c * N_NODES, dtype=jnp.int32)
    for r in range(BLK):
      for g in range(8):
        sl = pl.ds(g * 16, 16)
        idx_b[qn * 12 + r, sl] = idx_b[qn * 12 + r, sl] + coff

  def spmm(gtab, pk):
    tb = s * NBLK
    pltpu.sync_copy(pk.at[tb], idx_b.at[pl.ds(0, 12)])
    idx_fix(0)
    pltpu.async_copy(gtab.at[idx_b.at[0]], rv.at[0], gsem)
    pltpu.async_copy(gtab.at[idx_b.at[1]], rv.at[1], gsem)

    def step(cj, _):
      blk = cj // 4
      q = lax.rem(blk, 2)
      r = lax.rem(cj, 4)
      @pl.when(cj >= 2)
      def _():
        cp = cj - 2
        rp = lax.rem(cp, 4)
        qp = lax.rem(cp // 4, 2)
        pltpu.make_async_copy(rv.at[rp], accum.at[idx_b.at[qp * 12 + 4 + rp]],
                              ssem).wait()
      @pl.when((r == 1) & (blk + 1 < NBLK))
      def _():
        qn = lax.rem(blk + 1, 2)
        pltpu.sync_copy(pk.at[tb + blk + 1], idx_b.at[pl.ds(qn * 12, 12)])
        idx_fix(qn)
      @pl.when(cj + 2 < KPT)
      def _():
        cn = cj + 2
        rn = lax.rem(cn, 4)
        qn2 = lax.rem(cn // 4, 2)
        pltpu.async_copy(gtab.at[idx_b.at[qn2 * 12 + rn]], rv.at[rn], gsem)
      pltpu.make_async_copy(gtab.at[idx_b.at[q * 12 + r]], rv.at[r],
                            gsem).wait()
      vrow = q * 12 + 8 + r
      for g in range(8):
        vv = plsc.bitcast(idx_b[vrow, pl.ds(g * 16, 16)], jnp.float32)
        for e in range(16):
          i = g * 16 + e
          vb = _bcast16(vv, e)
          for h in range(2):
            sl = pl.ds(h * 16, 16)
            rv[r, i, sl] = rv[r, i, sl] * vb
      pltpu.async_copy(rv.at[r], accum.at[idx_b.at[q * 12 + 4 + r]], ssem,
                       add=True)
      return 0
    lax.fori_loop(0, KPT, step, 0)
    pltpu.make_async_copy(rv.at[2], accum.at[idx_b.at[12 + 4 + 2]],
                          ssem).wait()
    pltpu.make_async_copy(rv.at[3], accum.at[idx_b.at[12 + 4 + 3]],
                          ssem).wait()

  def writeout_t():
    def w(wi, _):
      cid = wi * NS + s
      @pl.when(cid < NWC)
      def _():
        rb = cid * WCH
        pltpu.sync_copy(accum.at[pl.ds(rb, WCH)], acc_v)
        pltpu.sync_copy(acc_v, tbuf.at[pl.ds(c * N_NODES + rb, WCH)])
        pltpu.sync_copy(zero_v, accum.at[pl.ds(rb, WCH)])
      return 0
    lax.fori_loop(0, NWT, w, 0)

  def writeout_ego(k):
    slope = jnp.where(k == 1, 1.0, LEAKY_SLOPE).astype(jnp.float32)
    sv = jnp.full((16,), slope, dtype=jnp.float32)

    def w(wi, _):
      cid = wi * NS + s
      @pl.when(cid < NWC)
      def _():
        rb = cid * WCH
        hb = c * N_NODES + rb
        pltpu.sync_copy(accum.at[pl.ds(rb, WCH)], acc_v)
        pltpu.sync_copy(ego2.at[pl.ds(hb, WCH)], prev_v)

        def rowf(i, _):
          for h in range(2):
            sl = pl.ds(h * 16, 16)
            x = acc_v[i, sl]
            x = jnp.where(x >= 0, x, x * sv)
            acc_v[i, sl] = x + prev_v[i, sl]
          return 0
        lax.fori_loop(0, WCH, rowf, 0)

        pltpu.sync_copy(acc_v, ego2.at[pl.ds(hb, WCH)])
        pltpu.sync_copy(zero_v, accum.at[pl.ds(rb, WCH)])
      return 0
    lax.fori_loop(0, NWT, w, 0)

  def layer(k, _):
    spmm(ego2, pk1)
    plsc.subcore_barrier()
    writeout_t()
    plsc.subcore_barrier()
    spmm(tbuf, pk2)
    plsc.subcore_barrier()
    writeout_ego(k)
    plsc.subcore_barrier()
    return 0
  lax.fori_loop(0, 2, layer, 0)


_TAB = jax.ShapeDtypeStruct((2 * N_NODES, DH), jnp.float32)
_propagate = pl.kernel(
    _prop_body,
    out_type=(_TAB, _TAB),
    mesh=plsc.VectorSubcoreMesh(core_axis_name="c", subcore_axis_name="s"),
    compiler_params=pltpu.CompilerParams(use_tc_tiling_on_sc=False,
                                         needs_layout_passes=False),
    scratch_types=[
        pltpu.VMEM_SHARED((N_NODES, DH), jnp.float32),
        pltpu.VMEM((24, CHUNK), jnp.int32),
        pltpu.VMEM((4, CHUNK, DH), jnp.float32),
        pltpu.VMEM((WCH, DH), jnp.float32),
        pltpu.VMEM((WCH, DH), jnp.float32),
        pltpu.VMEM((WCH, DH), jnp.float32),
        pltpu.SemaphoreType.DMA,
        pltpu.SemaphoreType.DMA,
    ],
)


def _mm_body(x_ref, w_ref, o_ref):
  hb = pl.program_id(0)
  ib = pl.program_id(1)
  w = jnp.where(ib < N_USERS // 1000, w_ref[0], w_ref[1])
  wh = jnp.where(hb == 0, w[:, :DH], w[:, DH:])
  o_ref[...] = jnp.dot(x_ref[...], wh, preferred_element_type=jnp.float32)


def _ego_flat(user_emb, item_emb, user_w, item_w):
  x = jnp.concatenate([user_emb, item_emb], axis=0)
  w2 = jnp.stack([user_w, item_w])
  bm = 1000
  return pl.pallas_call(
      _mm_body,
      grid=(2, N_NODES // bm),
      in_specs=[
          pl.BlockSpec((bm, DIM), lambda h, i: (i, 0)),
          pl.BlockSpec((2, DIM, DIM), lambda h, i: (0, 0, 0)),
      ],
      out_specs=pl.BlockSpec((bm, DH),
                             lambda h, i: (h * (N_NODES // bm) + i, 0)),
      out_shape=jax.ShapeDtypeStruct((2 * N_NODES, DH), jnp.float32),
  )(x, w2)


def _pack(src, dst, vals):
  vi = lax.bitcast_convert_type(vals, jnp.int32)
  s3 = src.reshape(NBT, BLK, CHUNK)
  d3 = dst.reshape(NBT, BLK, CHUNK)
  v3 = vi.reshape(NBT, BLK, CHUNK)
  return jnp.concatenate([s3, d3, v3], axis=1)


def kernel(user_emb, item_emb, user_w, item_w, adj_values, adj_indices,
           keep_rate=1):
  ego_flat = _ego_flat(user_emb, item_emb, user_w, item_w)

  rows = adj_indices[0].astype(jnp.int32)
  cols = adj_indices[1].astype(jnp.int32)
  pad = EP - E_EDGES
  pad_idx = (jnp.arange(pad, dtype=jnp.int32) * 61) % N_NODES
  rows_p = jnp.concatenate([rows, pad_idx])
  cols_p = jnp.concatenate([cols, pad_idx])
  vals_p = jnp.concatenate([adj_values, jnp.zeros((pad,), jnp.float32)])

  pk1 = _pack(rows_p, cols_p, vals_p)
  pk2 = _pack(cols_p, rows_p, vals_p)

  ego2, _ = _propagate(ego_flat, pk1, pk2)
  full = jnp.concatenate([ego2[:N_NODES], ego2[N_NODES:]], axis=1)
  return full[:N_USERS], full[N_USERS:]


# speedup vs baseline: 1.5346x; 1.0124x over previous
import jax
import jax.numpy as jnp
from jax import lax
from jax.experimental import pallas as pl
from jax.experimental.pallas import tpu as pltpu
from jax.experimental.pallas import tpu_sc as plsc

N_USERS = 25000
N_ITEMS = 25000
N_NODES = N_USERS + N_ITEMS
E_EDGES = 800000
DIM = 64
DH = 32
LEAKY_SLOPE = 0.2
NC = 2
NS = 16
CHUNK = 128
KPT = 400
KC = KPT * NS
EP = KC * CHUNK
BLK = 4
NBLK = KPT // BLK
PAIRS = NBLK // 2
NBT = KC // BLK
WCH = 80
NWC = N_NODES // WCH
NWT = (NWC + NS - 1) // NS


_GD = lax.GatherDimensionNumbers(offset_dims=(), collapsed_slice_dims=(0,),
                                 start_index_map=(0,))


def _bcast16(vec, e):
  idx = jnp.full((16, 1), e, dtype=jnp.int32)
  return lax.gather(vec, idx, _GD, (1,),
                    mode=lax.GatherScatterMode.PROMISE_IN_BOUNDS)


def _prop_body(ego0, pk,
               ego2, tbuf,
               accum, idx_b, rv, acc_v, prev_v, zero_v,
               gsem, ssem):
  c = lax.axis_index("c")
  s = lax.axis_index("s")

  def zrow(i, _):
    zero_v[i, pl.ds(0, 16)] = jnp.zeros((16,), jnp.float32)
    zero_v[i, pl.ds(16, 16)] = jnp.zeros((16,), jnp.float32)
    return 0
  lax.fori_loop(0, WCH, zrow, 0)

  def init_w(w, _):
    cid = w * NS + s
    @pl.when(cid < NWC)
    def _():
      rb = cid * WCH
      pltpu.sync_copy(zero_v, accum.at[pl.ds(rb, WCH)])
      pltpu.sync_copy(ego0.at[pl.ds(c * N_NODES + rb, WCH)], acc_v)
      pltpu.sync_copy(acc_v, ego2.at[pl.ds(c * N_NODES + rb, WCH)])
    return 0
  lax.fori_loop(0, NWT, init_w, 0)
  plsc.subcore_barrier()

  def idx_fix(qn, go):
    coff = jnp.full((16,), c * N_NODES, dtype=jnp.int32)
    for r in range(BLK):
      for g in range(8):
        sl = pl.ds(g * 16, 16)
        idx_b[qn * 12 + go + r, sl] = idx_b[qn * 12 + go + r, sl] + coff

  def spmm(gtab, go, so):
    tb = s * NBLK
    pltpu.sync_copy(pk.at[tb], idx_b.at[pl.ds(0, 12)])
    idx_fix(0, go)
    pltpu.async_copy(gtab.at[idx_b.at[go]], rv.at[0], gsem)
    pltpu.async_copy(gtab.at[idx_b.at[go + 1]], rv.at[1], gsem)

    def step(cj, _):
      blk = cj // 4
      q = lax.rem(blk, 2)
      r = lax.rem(cj, 4)
      @pl.when(cj >= 2)
      def _():
        cp = cj - 2
        rp = lax.rem(cp, 4)
        qp = lax.rem(cp // 4, 2)
        pltpu.make_async_copy(rv.at[rp],
                              accum.at[idx_b.at[qp * 12 + so + rp]],
                              ssem).wait()
      @pl.when((r == 1) & (blk + 1 < NBLK))
      def _():
        qn = lax.rem(blk + 1, 2)
        pltpu.sync_copy(pk.at[tb + blk + 1], idx_b.at[pl.ds(qn * 12, 12)])
        idx_fix(qn, go)
      @pl.when(cj + 2 < KPT)
      def _():
        cn = cj + 2
        rn = lax.rem(cn, 4)
        qn2 = lax.rem(cn // 4, 2)
        pltpu.async_copy(gtab.at[idx_b.at[qn2 * 12 + go + rn]], rv.at[rn],
                         gsem)
      pltpu.make_async_copy(gtab.at[idx_b.at[q * 12 + go + r]], rv.at[r],
                            gsem).wait()
      vrow = q * 12 + 8 + r
      for g in range(8):
        vv = plsc.bitcast(idx_b[vrow, pl.ds(g * 16, 16)], jnp.float32)
        for e in range(16):
          i = g * 16 + e
          vb = _bcast16(vv, e)
          for h in range(2):
            sl = pl.ds(h * 16, 16)
            rv[r, i, sl] = rv[r, i, sl] * vb
      pltpu.async_copy(rv.at[r], accum.at[idx_b.at[q * 12 + so + r]], ssem,
                       add=True)
      return 0
    lax.fori_loop(0, KPT, step, 0)
    pltpu.make_async_copy(rv.at[2], accum.at[idx_b.at[12 + so + 2]],
                          ssem).wait()
    pltpu.make_async_copy(rv.at[3], accum.at[idx_b.at[12 + so + 3]],
                          ssem).wait()

  def writeout_t():
    def w(wi, _):
      cid = wi * NS + s
      @pl.when(cid < NWC)
      def _():
        rb = cid * WCH
        pltpu.sync_copy(accum.at[pl.ds(rb, WCH)], acc_v)
        pltpu.sync_copy(acc_v, tbuf.at[pl.ds(c * N_NODES + rb, WCH)])
        pltpu.sync_copy(zero_v, accum.at[pl.ds(rb, WCH)])
      return 0
    lax.fori_loop(0, NWT, w, 0)

  def writeout_ego(k):
    slope = jnp.where(k == 1, 1.0, LEAKY_SLOPE).astype(jnp.float32)
    sv = jnp.full((16,), slope, dtype=jnp.float32)

    def w(wi, _):
      cid = wi * NS + s
      @pl.when(cid < NWC)
      def _():
        rb = cid * WCH
        hb = c * N_NODES + rb
        pltpu.sync_copy(accum.at[pl.ds(rb, WCH)], acc_v)
        pltpu.sync_copy(ego2.at[pl.ds(hb, WCH)], prev_v)

        def rowf(i, _):
          for h in range(2):
            sl = pl.ds(h * 16, 16)
            x = acc_v[i, sl]
            x = jnp.where(x >= 0, x, x * sv)
            acc_v[i, sl] = x + prev_v[i, sl]
          return 0
        lax.fori_loop(0, WCH, rowf, 0)

        pltpu.sync_copy(acc_v, ego2.at[pl.ds(hb, WCH)])
        pltpu.sync_copy(zero_v, accum.at[pl.ds(rb, WCH)])
      return 0
    lax.fori_loop(0, NWT, w, 0)

  def layer(k, _):
    spmm(ego2, 0, 4)
    plsc.subcore_barrier()
    writeout_t()
    plsc.subcore_barrier()
    spmm(tbuf, 4, 0)
    plsc.subcore_barrier()
    writeout_ego(k)
    plsc.subcore_barrier()
    return 0
  lax.fori_loop(0, 2, layer, 0)


_TAB = jax.ShapeDtypeStruct((2 * N_NODES, DH), jnp.float32)
_propagate = pl.kernel(
    _prop_body,
    out_type=(_TAB, _TAB),
    mesh=plsc.VectorSubcoreMesh(core_axis_name="c", subcore_axis_name="s"),
    compiler_params=pltpu.CompilerParams(use_tc_tiling_on_sc=False,
                                         needs_layout_passes=False),
    scratch_types=[
        pltpu.VMEM_SHARED((N_NODES, DH), jnp.float32),
        pltpu.VMEM((24, CHUNK), jnp.int32),
        pltpu.VMEM((4, CHUNK, DH), jnp.float32),
        pltpu.VMEM((WCH, DH), jnp.float32),
        pltpu.VMEM((WCH, DH), jnp.float32),
        pltpu.VMEM((WCH, DH), jnp.float32),
        pltpu.SemaphoreType.DMA,
        pltpu.SemaphoreType.DMA,
    ],
)


def _mm_body(x_ref, w_ref, o_ref):
  hb = pl.program_id(0)
  ib = pl.program_id(1)
  w = jnp.where(ib < N_USERS // 1000, w_ref[0], w_ref[1])
  wh = jnp.where(hb == 0, w[:, :DH], w[:, DH:])
  o_ref[...] = jnp.dot(x_ref[...], wh, preferred_element_type=jnp.float32)


def _ego_flat(user_emb, item_emb, user_w, item_w):
  x = jnp.concatenate([user_emb, item_emb], axis=0)
  w2 = jnp.stack([user_w, item_w])
  bm = 1000
  return pl.pallas_call(
      _mm_body,
      grid=(2, N_NODES // bm),
      in_specs=[
          pl.BlockSpec((bm, DIM), lambda h, i: (i, 0)),
          pl.BlockSpec((2, DIM, DIM), lambda h, i: (0, 0, 0)),
      ],
      out_specs=pl.BlockSpec((bm, DH),
                             lambda h, i: (h * (N_NODES // bm) + i, 0)),
      out_shape=jax.ShapeDtypeStruct((2 * N_NODES, DH), jnp.float32),
  )(x, w2)


def _pack(src, dst, vals):
  vi = lax.bitcast_convert_type(vals, jnp.int32)
  s3 = src.reshape(NBT, BLK, CHUNK)
  d3 = dst.reshape(NBT, BLK, CHUNK)
  v3 = vi.reshape(NBT, BLK, CHUNK)
  return jnp.concatenate([s3, d3, v3], axis=1)


def kernel(user_emb, item_emb, user_w, item_w, adj_values, adj_indices,
           keep_rate=1):
  ego_flat = _ego_flat(user_emb, item_emb, user_w, item_w)

  rows = adj_indices[0].astype(jnp.int32)
  cols = adj_indices[1].astype(jnp.int32)
  pad = EP - E_EDGES
  pad_idx = (jnp.arange(pad, dtype=jnp.int32) * 61) % N_NODES
  rows_p = jnp.concatenate([rows, pad_idx])
  cols_p = jnp.concatenate([cols, pad_idx])
  vals_p = jnp.concatenate([adj_values, jnp.zeros((pad,), jnp.float32)])

  pk = _pack(rows_p, cols_p, vals_p)

  ego2, _ = _propagate(ego_flat, pk)
  full = jnp.concatenate([ego2[:N_NODES], ego2[N_NODES:]], axis=1)
  return full[:N_USERS], full[N_USERS:]
